# speedup vs baseline: 3.9306x; 3.9306x over previous
import functools

import jax
import jax.numpy as jnp
from jax import lax
from jax.experimental import pallas as pl
from jax.experimental.pallas import tpu as pltpu
from jax.experimental.pallas import tpu_sc as plsc

ATOM_FEA_LEN = 128
NUM_RADIAL = 16
OUT_LEN = 128
N_NODES = 10000
N_EDGES = 320000

_NC = 2
_NS = 16
_NW = _NC * _NS
_E_PER_W = N_EDGES // _NW
_CHUNK = 80
_N_CHUNKS = _E_PER_W // _CHUNK
_SEG = 16
_SEGS = OUT_LEN // _SEG

_RBF_BLK = 1280


def _matmul_body(x_ref, w_ref, o_ref):
    o_ref[...] = jnp.dot(
        x_ref[...], w_ref[...],
        preferred_element_type=jnp.float32,
        precision=lax.Precision.HIGHEST,
    )


def _node_table(atom_fea, w01):
    return pl.pallas_call(
        _matmul_body,
        grid=(2,),
        in_specs=[
            pl.BlockSpec((N_NODES, ATOM_FEA_LEN), lambda t: (0, 0)),
            pl.BlockSpec((ATOM_FEA_LEN, OUT_LEN), lambda t: (t, 0)),
        ],
        out_specs=pl.BlockSpec((N_NODES, OUT_LEN), lambda t: (t, 0)),
        out_shape=jax.ShapeDtypeStruct((2 * N_NODES, OUT_LEN), jnp.float32),
    )(atom_fea, w01)


def _rbf_proj(rbf, w2):
    return pl.pallas_call(
        _matmul_body,
        grid=(N_EDGES // _RBF_BLK,),
        in_specs=[
            pl.BlockSpec((_RBF_BLK, NUM_RADIAL), lambda t: (t, 0)),
            pl.BlockSpec((NUM_RADIAL, OUT_LEN), lambda t: (0, 0)),
        ],
        out_specs=pl.BlockSpec((_RBF_BLK, OUT_LEN), lambda t: (t, 0)),
        out_shape=jax.ShapeDtypeStruct((N_EDGES, OUT_LEN), jnp.float32),
    )(rbf, w2)


def _sc_body(t_hbm, i0_hbm, i1_hbm, r_hbm, out_hbm,
             i0_v, i1_v, g0_v, g1_v, r_v, sem0, sem1):
    wid = lax.axis_index("s") * _NC + lax.axis_index("c")
    base0 = wid * _E_PER_W

    def chunk_body(ci, carry):
        base = base0 + ci * _CHUNK
        pltpu.sync_copy(i0_hbm.at[pl.ds(base, _CHUNK)], i0_v)
        pltpu.sync_copy(i1_hbm.at[pl.ds(base, _CHUNK)], i1_v)
        c0 = pltpu.async_copy(t_hbm.at[i0_v], g0_v, sem0)
        c1 = pltpu.async_copy(t_hbm.at[i1_v], g1_v, sem1)
        pltpu.sync_copy(r_hbm.at[pl.ds(base, _CHUNK)], r_v)
        c0.wait()
        c1.wait()

        def row_body(r, c2):
            for c in range(_SEGS):
                sl = pl.ds(c * _SEG, _SEG)
                g0_v[r, sl] = g0_v[r, sl] + g1_v[r, sl] + r_v[r, sl]
            return c2

        lax.fori_loop(0, _CHUNK, row_body, 0)
        pltpu.sync_copy(g0_v, out_hbm.at[pl.ds(base, _CHUNK)])
        return carry

    lax.fori_loop(0, _N_CHUNKS, chunk_body, 0)


@functools.partial(
    pl.kernel,
    out_type=jax.ShapeDtypeStruct((N_EDGES, OUT_LEN), jnp.float32),
    mesh=plsc.VectorSubcoreMesh(core_axis_name="c", subcore_axis_name="s"),
    scratch_types=[
        pltpu.VMEM((_CHUNK,), jnp.int32),
        pltpu.VMEM((_CHUNK,), jnp.int32),
        pltpu.VMEM((_CHUNK, OUT_LEN), jnp.float32),
        pltpu.VMEM((_CHUNK, OUT_LEN), jnp.float32),
        pltpu.VMEM((_CHUNK, OUT_LEN), jnp.float32),
        pltpu.SemaphoreType.DMA,
        pltpu.SemaphoreType.DMA,
    ],
)
def _sc_combine(t_hbm, i0_hbm, i1_hbm, r_hbm, out_hbm, *scratch):
    _sc_body(t_hbm, i0_hbm, i1_hbm, r_hbm, out_hbm, *scratch)


def kernel(atom_fea, rbf, nbr_fea_idx, W):
    w01 = W[: 2 * ATOM_FEA_LEN]
    w2 = W[2 * ATOM_FEA_LEN :]
    table = _node_table(atom_fea, w01)
    r = _rbf_proj(rbf, w2)
    i0 = nbr_fea_idx[:, 0]
    i1 = nbr_fea_idx[:, 1] + N_NODES
    return _sc_combine(table, i0, i1, r)


# speedup vs baseline: 6.6343x; 1.6878x over previous
import functools

import jax
import jax.numpy as jnp
from jax import lax
from jax.experimental import pallas as pl
from jax.experimental.pallas import tpu as pltpu
from jax.experimental.pallas import tpu_sc as plsc

ATOM_FEA_LEN = 128
NUM_RADIAL = 16
OUT_LEN = 128
N_NODES = 10000
N_EDGES = 320000

_NC = 2
_NS = 16
_NW = _NC * _NS
_E_PER_W = N_EDGES // _NW
_CHUNK = 80
_N_CHUNKS = _E_PER_W // _CHUNK
_N_PAIRS = (_N_CHUNKS - 1) // 2
_SEG = 16
_SEGS = OUT_LEN // _SEG

_RBF_BLK = 3200


def _matmul_body(x_ref, w_ref, o_ref):
    o_ref[...] = jnp.dot(
        x_ref[...], w_ref[...],
        preferred_element_type=jnp.float32,
        precision=lax.Precision.HIGHEST,
    )


def _node_table(atom_fea, w01):
    return pl.pallas_call(
        _matmul_body,
        grid=(2,),
        in_specs=[
            pl.BlockSpec((N_NODES, ATOM_FEA_LEN), lambda t: (0, 0)),
            pl.BlockSpec((ATOM_FEA_LEN, OUT_LEN), lambda t: (t, 0)),
        ],
        out_specs=pl.BlockSpec((N_NODES, OUT_LEN), lambda t: (t, 0)),
        out_shape=jax.ShapeDtypeStruct((2 * N_NODES, OUT_LEN), jnp.float32),
    )(atom_fea, w01)


def _rbf_proj(rbf, w2):
    return pl.pallas_call(
        _matmul_body,
        grid=(N_EDGES // _RBF_BLK,),
        in_specs=[
            pl.BlockSpec((_RBF_BLK, NUM_RADIAL), lambda t: (t, 0)),
            pl.BlockSpec((NUM_RADIAL, OUT_LEN), lambda t: (0, 0)),
        ],
        out_specs=pl.BlockSpec((_RBF_BLK, OUT_LEN), lambda t: (t, 0)),
        out_shape=jax.ShapeDtypeStruct((N_EDGES, OUT_LEN), jnp.float32),
    )(rbf, w2)


def _sc_body(t_hbm, i0_hbm, i1_hbm, r_hbm, out_hbm,
             i0_v, i1_v, g0_v, g1_v, r_v, sem0, sem1):
    sems = (sem0, sem1)
    wid = lax.axis_index("s") * _NC + lax.axis_index("c")
    base0 = wid * _E_PER_W

    pltpu.sync_copy(i0_hbm.at[pl.ds(base0, _E_PER_W)], i0_v)
    pltpu.sync_copy(i1_hbm.at[pl.ds(base0, _E_PER_W)], i1_v)

    def issue(b, ci):
        off = ci * _CHUNK
        pltpu.async_copy(t_hbm.at[i0_v.at[pl.ds(off, _CHUNK)]], g0_v.at[b],
                         sems[b])
        pltpu.async_copy(t_hbm.at[i1_v.at[pl.ds(off, _CHUNK)]], g1_v.at[b],
                         sems[b])
        pltpu.async_copy(r_hbm.at[pl.ds(base0 + off, _CHUNK)], r_v.at[b],
                         sems[b])

    def drain(b):
        dummy = r_hbm.at[pl.ds(0, _CHUNK)]
        pltpu.make_async_copy(dummy, g0_v.at[b], sems[b]).wait()
        pltpu.make_async_copy(dummy, g1_v.at[b], sems[b]).wait()
        pltpu.make_async_copy(dummy, r_v.at[b], sems[b]).wait()

    def combine_and_store(b, ci):
        def row_body(r, carry):
            for c in range(_SEGS):
                sl = pl.ds(c * _SEG, _SEG)
                g0_v[b, r, sl] = g0_v[b, r, sl] + g1_v[b, r, sl] + r_v[b, r, sl]
            return carry

        lax.fori_loop(0, _CHUNK, row_body, 0)
        pltpu.sync_copy(g0_v.at[b], out_hbm.at[pl.ds(base0 + ci * _CHUNK,
                                                     _CHUNK)])

    issue(0, 0)

    def pair_body(p, carry):
        issue(1, 2 * p + 1)
        drain(0)
        combine_and_store(0, 2 * p)
        issue(0, 2 * p + 2)
        drain(1)
        combine_and_store(1, 2 * p + 1)
        return carry

    lax.fori_loop(0, _N_PAIRS, pair_body, 0)
    drain(0)
    combine_and_store(0, _N_CHUNKS - 1)


@functools.partial(
    pl.kernel,
    out_type=jax.ShapeDtypeStruct((N_EDGES, OUT_LEN), jnp.float32),
    mesh=plsc.VectorSubcoreMesh(core_axis_name="c", subcore_axis_name="s"),
    scratch_types=[
        pltpu.VMEM((_E_PER_W,), jnp.int32),
        pltpu.VMEM((_E_PER_W,), jnp.int32),
        pltpu.VMEM((2, _CHUNK, OUT_LEN), jnp.float32),
        pltpu.VMEM((2, _CHUNK, OUT_LEN), jnp.float32),
        pltpu.VMEM((2, _CHUNK, OUT_LEN), jnp.float32),
        pltpu.SemaphoreType.DMA,
        pltpu.SemaphoreType.DMA,
    ],
)
def _sc_combine(t_hbm, i0_hbm, i1_hbm, r_hbm, out_hbm, *scratch):
    _sc_body(t_hbm, i0_hbm, i1_hbm, r_hbm, out_hbm, *scratch)


def kernel(atom_fea, rbf, nbr_fea_idx, W):
    w01 = W[: 2 * ATOM_FEA_LEN]
    w2 = W[2 * ATOM_FEA_LEN :]
    table = _node_table(atom_fea, w01)
    r = _rbf_proj(rbf, w2)
    i0 = nbr_fea_idx[:, 0]
    i1 = nbr_fea_idx[:, 1] + N_NODES
    return _sc_combine(table, i0, i1, r)


# speedup vs baseline: 7.6653x; 1.1554x over previous
import functools

import jax
import jax.numpy as jnp
from jax import lax
from jax.experimental import pallas as pl
from jax.experimental.pallas import tpu as pltpu
from jax.experimental.pallas import tpu_sc as plsc

ATOM_FEA_LEN = 128
NUM_RADIAL = 16
OUT_LEN = 128
N_NODES = 10000
N_EDGES = 320000

_NC = 2
_NS = 16
_NW = _NC * _NS
_E_PER_W = N_EDGES // _NW
_CHUNK = 80
_N_CHUNKS = _E_PER_W // _CHUNK
_N_PAIRS = (_N_CHUNKS - 1) // 2
_SEG = 16
_SEGS = OUT_LEN // _SEG

_RBF_BLK = 3200


def _matmul_body(x_ref, w_ref, o_ref):
    o_ref[...] = jnp.dot(
        x_ref[...], w_ref[...],
        preferred_element_type=jnp.float32,
        precision=lax.Precision.HIGHEST,
    )


def _matmul_body_bf16(x_ref, w_ref, o_ref):
    o_ref[...] = jnp.dot(
        x_ref[...], w_ref[...],
        preferred_element_type=jnp.float32,
    )


def _node_table(atom_fea, w01):
    return pl.pallas_call(
        _matmul_body,
        grid=(2,),
        in_specs=[
            pl.BlockSpec((N_NODES, ATOM_FEA_LEN), lambda t: (0, 0)),
            pl.BlockSpec((ATOM_FEA_LEN, OUT_LEN), lambda t: (t, 0)),
        ],
        out_specs=pl.BlockSpec((N_NODES, OUT_LEN), lambda t: (t, 0)),
        out_shape=jax.ShapeDtypeStruct((2 * N_NODES, OUT_LEN), jnp.float32),
    )(atom_fea, w01)


def _rbf_proj(rbf, w2):
    return pl.pallas_call(
        _matmul_body_bf16,
        grid=(N_EDGES // _RBF_BLK,),
        in_specs=[
            pl.BlockSpec((_RBF_BLK, NUM_RADIAL), lambda t: (t, 0)),
            pl.BlockSpec((NUM_RADIAL, OUT_LEN), lambda t: (0, 0)),
        ],
        out_specs=pl.BlockSpec((_RBF_BLK, OUT_LEN), lambda t: (t, 0)),
        out_shape=jax.ShapeDtypeStruct((N_EDGES, OUT_LEN), jnp.float32),
    )(rbf.astype(jnp.bfloat16), w2.astype(jnp.bfloat16))


def _sc_body(t_hbm, i0_hbm, i1_hbm, r_hbm, out_hbm,
             i0_v, i1_v, g0_v, g1_v, r_v, sem0, sem1):
    sems = (sem0, sem1)
    wid = lax.axis_index("s") * _NC + lax.axis_index("c")
    base0 = wid * _E_PER_W

    pltpu.sync_copy(i0_hbm.at[pl.ds(base0, _E_PER_W)], i0_v)
    pltpu.sync_copy(i1_hbm.at[pl.ds(base0, _E_PER_W)], i1_v)

    def issue(b, ci):
        off = ci * _CHUNK
        pltpu.async_copy(t_hbm.at[i0_v.at[pl.ds(off, _CHUNK)]], g0_v.at[b],
                         sems[b])
        pltpu.async_copy(t_hbm.at[i1_v.at[pl.ds(off, _CHUNK)]], g1_v.at[b],
                         sems[b])
        pltpu.async_copy(r_hbm.at[pl.ds(base0 + off, _CHUNK)], r_v.at[b],
                         sems[b])

    def drain(b):
        dummy = r_hbm.at[pl.ds(0, _CHUNK)]
        pltpu.make_async_copy(dummy, g0_v.at[b], sems[b]).wait()
        pltpu.make_async_copy(dummy, g1_v.at[b], sems[b]).wait()
        pltpu.make_async_copy(dummy, r_v.at[b], sems[b]).wait()

    def combine_and_store(b, ci):
        def row_body(r, carry):
            for c in range(_SEGS):
                sl = pl.ds(c * _SEG, _SEG)
                g0_v[b, r, sl] = g0_v[b, r, sl] + g1_v[b, r, sl] + r_v[b, r, sl]
            return carry

        lax.fori_loop(0, _CHUNK, row_body, 0)
        pltpu.sync_copy(g0_v.at[b], out_hbm.at[pl.ds(base0 + ci * _CHUNK,
                                                     _CHUNK)])

    issue(0, 0)

    def pair_body(p, carry):
        issue(1, 2 * p + 1)
        drain(0)
        combine_and_store(0, 2 * p)
        issue(0, 2 * p + 2)
        drain(1)
        combine_and_store(1, 2 * p + 1)
        return carry

    lax.fori_loop(0, _N_PAIRS, pair_body, 0)
    drain(0)
    combine_and_store(0, _N_CHUNKS - 1)


@functools.partial(
    pl.kernel,
    out_type=jax.ShapeDtypeStruct((N_EDGES, OUT_LEN), jnp.float32),
    mesh=plsc.VectorSubcoreMesh(core_axis_name="c", subcore_axis_name="s"),
    scratch_types=[
        pltpu.VMEM((_E_PER_W,), jnp.int32),
        pltpu.VMEM((_E_PER_W,), jnp.int32),
        pltpu.VMEM((2, _CHUNK, OUT_LEN), jnp.float32),
        pltpu.VMEM((2, _CHUNK, OUT_LEN), jnp.float32),
        pltpu.VMEM((2, _CHUNK, OUT_LEN), jnp.float32),
        pltpu.SemaphoreType.DMA,
        pltpu.SemaphoreType.DMA,
    ],
)
def _sc_combine(t_hbm, i0_hbm, i1_hbm, r_hbm, out_hbm, *scratch):
    _sc_body(t_hbm, i0_hbm, i1_hbm, r_hbm, out_hbm, *scratch)


def kernel(atom_fea, rbf, nbr_fea_idx, W):
    w01 = W[: 2 * ATOM_FEA_LEN]
    w2 = W[2 * ATOM_FEA_LEN :]
    table = _node_table(atom_fea, w01)
    r = _rbf_proj(rbf, w2)
    i0 = nbr_fea_idx[:, 0]
    i1 = nbr_fea_idx[:, 1] + N_NODES
    return _sc_combine(table, i0, i1, r)
